# baseline (speedup 1.0000x reference)
import functools

import jax
import jax.numpy as jnp
from jax import lax
from jax.experimental import pallas as pl
from jax.experimental.pallas import tpu as pltpu
from jax.experimental.pallas import tpu_sc as plsc


def kernel(input):
    shape = input.shape
    n = input.size
    x1d = input.reshape(n)

    info = plsc.get_sparse_core_info()
    nc, ns = info.num_cores, info.num_subcores
    nw = nc * ns
    chunk = n // nw

    mesh = plsc.VectorSubcoreMesh(core_axis_name="c", subcore_axis_name="s")

    @functools.partial(
        pl.kernel,
        mesh=mesh,
        out_type=jax.ShapeDtypeStruct((n,), input.dtype),
    )
    def sc_copy(x_hbm, o_hbm):
        wid = lax.axis_index("s") * nc + lax.axis_index("c")
        base = wid * chunk
        pltpu.sync_copy(x_hbm.at[pl.ds(base, chunk)],
                        o_hbm.at[pl.ds(base, chunk)])

    return sc_copy(x1d).reshape(shape)


# speedup vs baseline: 24.8613x; 24.8613x over previous
import jax
import jax.numpy as jnp
from jax.experimental import pallas as pl
from jax.experimental.pallas import tpu as pltpu


def _copy_kernel(x_ref, o_ref, buf0, buf1, si0, si1, so0, so1):
    h = x_ref.shape[0] // 2
    in0 = pltpu.make_async_copy(x_ref.at[pl.ds(0, h)], buf0, si0)
    in1 = pltpu.make_async_copy(x_ref.at[pl.ds(h, h)], buf1, si1)
    in0.start()
    in1.start()
    in0.wait()
    out0 = pltpu.make_async_copy(buf0, o_ref.at[pl.ds(0, h)], so0)
    out0.start()
    in1.wait()
    out1 = pltpu.make_async_copy(buf1, o_ref.at[pl.ds(h, h)], so1)
    out1.start()
    out0.wait()
    out1.wait()


def kernel(input):
    t, b, s = input.shape
    return pl.pallas_call(
        _copy_kernel,
        out_shape=jax.ShapeDtypeStruct(input.shape, input.dtype),
        in_specs=[pl.BlockSpec(memory_space=pl.ANY)],
        out_specs=pl.BlockSpec(memory_space=pl.ANY),
        scratch_shapes=[
            pltpu.VMEM((t // 2, b, s), input.dtype),
            pltpu.VMEM((t // 2, b, s), input.dtype),
            pltpu.SemaphoreType.DMA,
            pltpu.SemaphoreType.DMA,
            pltpu.SemaphoreType.DMA,
            pltpu.SemaphoreType.DMA,
        ],
    )(input)
